# baseline (speedup 1.0000x reference)
import functools

import jax
import jax.numpy as jnp
from jax import lax
from jax.experimental import pallas as pl
from jax.experimental.pallas import tpu as pltpu
from jax.experimental.pallas import tpu_sc as plsc

N = 10000
E = 640000
H = 128
NPAD = 112
NA = N + NPAD
NC = 2
NS = 16
NW = NC * NS
CW = 128
WPW = 157
W_A = 80
W_B = WPW - W_A
EPA = W_A * CW
EPB = W_B * CW
E_A = NW * EPA
E_B = NW * EPB
E_PAD = E_A + E_B
NA_PER_S = NA // NS
NBUF = 4

MLP_TILE = 2048
TN = 1000

_mesh = plsc.VectorSubcoreMesh(core_axis_name="c", subcore_axis_name="s")


def _worker_id():
    return lax.axis_index("s") * NC + lax.axis_index("c")


def _goff(wid, gw):
    return jnp.where(gw < W_A,
                     wid * EPA + gw * CW,
                     E_A + wid * EPB + (gw - W_A) * CW)


def _make_sc_gather_x(w0: int, nwin: int, esize: int, delta: int):
    epw = nwin * CW

    @functools.partial(
        pl.kernel,
        mesh=_mesh,
        out_type=[
            jax.ShapeDtypeStruct((7, esize), jnp.float32),
            jax.ShapeDtypeStruct((7, esize), jnp.float32),
            jax.ShapeDtypeStruct((NW, 1, NA), jnp.float32),
        ],
        scratch_types=[
            pltpu.VMEM((NA * 7,), jnp.float32),
            pltpu.VMEM((epw,), jnp.int32),
            pltpu.VMEM((epw,), jnp.int32),
            pltpu.VMEM((7, CW), jnp.float32),
            pltpu.VMEM((7, CW), jnp.float32),
            pltpu.VMEM((7, CW), jnp.float32),
            pltpu.VMEM((7, CW), jnp.float32),
            pltpu.VMEM((NA,), jnp.float32),
            pltpu.SemaphoreType.DMA,
            pltpu.SemaphoreType.DMA,
            pltpu.SemaphoreType.DMA,
            pltpu.SemaphoreType.DMA,
        ],
        compiler_params=pltpu.CompilerParams(needs_layout_passes=False),
    )
    def gather_x(row_hbm, col_hbm, x7_hbm,
                 xc_hbm, xr_hbm, deg_hbm,
                 x7v, cidx, ridx, cb0, cb1, rb0, rb1, dacc,
                 cs0, cs1, rs0, rs1):
        wid = _worker_id()
        base = _goff(wid, w0)
        cbs = (cb0, cb1)
        rbs = (rb0, rb1)
        css = (cs0, cs1)
        rss = (rs0, rs1)

        def zbody(i, _):
            dacc[pl.ds(i * 16, 16)] = jnp.zeros((16,), jnp.float32)
            return _

        lax.fori_loop(0, NA // 16, zbody, None)
        pltpu.sync_copy(x7_hbm, x7v)
        pltpu.sync_copy(col_hbm.at[pl.ds(base, epw)], cidx)
        pltpu.sync_copy(row_hbm.at[pl.ds(base, epw)], ridx)
        ones16 = jnp.ones((16,), jnp.float32)

        def compute(w, b):
            for j in range(CW // 16):
                c16 = cidx[pl.ds(w * CW + j * 16, 16)]
                r16 = ridx[pl.ds(w * CW + j * 16, 16)]
                plsc.addupdate_scatter(dacc, [c16], ones16)
                c7 = c16 * 7
                r7 = r16 * 7
                for f in range(7):
                    cbs[b][f, pl.ds(j * 16, 16)] = plsc.load_gather(x7v, [c7 + f])
                    rbs[b][f, pl.ds(j * 16, 16)] = plsc.load_gather(x7v, [r7 + f])

        def issue_out(w, b):
            off = base - delta + w * CW
            pltpu.async_copy(cbs[b], xc_hbm.at[:, pl.ds(off, CW)], css[b])
            pltpu.async_copy(rbs[b], xr_hbm.at[:, pl.ds(off, CW)], rss[b])

        def wait_out(w, b):
            off = base - delta + w * CW
            pltpu.make_async_copy(cbs[b], xc_hbm.at[:, pl.ds(off, CW)], css[b]).wait()
            pltpu.make_async_copy(rbs[b], xr_hbm.at[:, pl.ds(off, CW)], rss[b]).wait()

        def body(k, _):
            for b in range(2):
                w = 2 * k + b

                @pl.when(w >= 2)
                def _():
                    wait_out(w - 2, b)

                compute(w, b)
                issue_out(w, b)
            return _

        lax.fori_loop(0, nwin // 2, body, None)
        if nwin % 2:
            w = nwin - 1
            wait_out(w - 2, 0)
            compute(w, 0)
            issue_out(w, 0)
        wait_out(nwin - 2, (nwin - 2) % 2)
        wait_out(nwin - 1, (nwin - 1) % 2)
        pltpu.sync_copy(dacc, deg_hbm.at[wid, 0])

    return gather_x


_sc_gather_x_a = _make_sc_gather_x(0, W_A, E_A, 0)
_sc_gather_x_b = _make_sc_gather_x(W_A, W_B, E_B, E_A)


def _make_sc_aggregate(gather_src: bool, w0: int = 0, nwin: int = WPW,
                       src_delta: int = 0, first: bool = True):
    scratch = [pltpu.VMEM((CW,), jnp.int32)] * 7
    scratch += [pltpu.VMEM((CW, H), jnp.float32)] * 3
    scratch += [pltpu.VMEM_SHARED((NA, H), jnp.float32)]
    scratch += [pltpu.SemaphoreType.DMA] * 13

    @functools.partial(
        pl.kernel,
        mesh=_mesh,
        out_type=jax.ShapeDtypeStruct((NC, NA, H), jnp.float32),
        scratch_types=scratch,
    )
    def agg(src_hbm, row_hbm, col_hbm, init_hbm, out_hbm,
            c0, c1, c2, c3, r0, r1, r2, b0, b1, b2, acc,
            ic0, ic1, ic2, ic3, ir0, ir1, ir2, g0, g1, g2, s0, s1, s2):
        cid = lax.axis_index("c")
        sid = lax.axis_index("s")
        wid = _worker_id()
        cidx = (c0, c1, c2, c3)
        ridx = (r0, r1, r2)
        bufs = (b0, b1, b2)
        icsem = (ic0, ic1, ic2, ic3)
        irsem = (ir0, ir1, ir2)
        gsem = (g0, g1, g2)
        ssem = (s0, s1, s2)

        rows = pl.ds(sid * NA_PER_S, NA_PER_S)
        if first:
            pltpu.sync_copy(init_hbm.at[rows], acc.at[rows])
        else:
            pltpu.sync_copy(init_hbm.at[cid, rows], acc.at[rows])
        plsc.subcore_barrier()

        def off(w):
            return _goff(wid, w0 + w)

        def issue_idx(w, ic, ir):
            o = off(w)
            pltpu.async_copy(col_hbm.at[pl.ds(o, CW)], cidx[ic], icsem[ic])
            if gather_src:
                pltpu.async_copy(row_hbm.at[pl.ds(o, CW)], ridx[ir], irsem[ir])

        def wait_idx(w, ic, ir):
            o = off(w)
            pltpu.make_async_copy(col_hbm.at[pl.ds(o, CW)], cidx[ic], icsem[ic]).wait()
            if gather_src:
                pltpu.make_async_copy(row_hbm.at[pl.ds(o, CW)], ridx[ir], irsem[ir]).wait()

        def gsrc(w, ir):
            if gather_src:
                return src_hbm.at[ridx[ir]]
            return src_hbm.at[pl.ds(off(w) - src_delta, CW)]

        def issue_gather(w, ir, b):
            pltpu.async_copy(gsrc(w, ir), bufs[b], gsem[b])

        def wait_gather(w, ir, b):
            pltpu.make_async_copy(gsrc(w, ir), bufs[b], gsem[b]).wait()

        def issue_scatter(w, ic, b):
            pltpu.async_copy(bufs[b], acc.at[cidx[ic]], ssem[b], add=True)

        def wait_scatter(w, ic, b):
            pltpu.make_async_copy(bufs[b], acc.at[cidx[ic]], ssem[b]).wait()

        issue_idx(0, 0, 0)
        issue_idx(1, 1, 1)

        def slot(w, j):
            @pl.when(w >= 2)
            def _():
                wait_scatter(w - 2, (j - 2) % 4, (j - 2) % 3)

            wait_idx(w, j % 4, j % 3)
            issue_gather(w, j % 3, j % 3)

            @pl.when(w >= 1)
            def _():
                wait_gather(w - 1, (j - 1) % 3, (j - 1) % 3)
                issue_scatter(w - 1, (j - 1) % 4, (j - 1) % 3)

            @pl.when(w + 2 < nwin)
            def _():
                issue_idx(w + 2, (j + 2) % 4, (j + 2) % 3)

        def body(k, _):
            w0_ = 12 * k
            for j in range(12):
                slot(w0_ + j, j)
            return _

        nfull = nwin // 12
        lax.fori_loop(0, nfull, body, None)
        for j in range(nwin % 12):
            slot(nfull * 12 + j, j)
        w = nwin - 1
        jw = w % 12
        wait_gather(w, jw % 3, jw % 3)
        issue_scatter(w, jw % 4, jw % 3)
        wait_scatter(w - 1, (jw - 1) % 4, (jw - 1) % 3)
        wait_scatter(w, jw % 4, jw % 3)

        plsc.subcore_barrier()
        pltpu.sync_copy(acc.at[rows], out_hbm.at[cid, rows])

    return agg


_sc_agg_lin_a = _make_sc_aggregate(False, 0, W_A, 0, first=True)
_sc_agg_lin_b = _make_sc_aggregate(False, W_A, W_B, E_A, first=False)
_sc_agg_gather = _make_sc_aggregate(True)


def _mlp_body(xc_ref, xr_ref, ea_ref, wc_ref, wr_ref, we_ref, b1_ref, out_ref):
    dn = (((0,), (0,)), ((), ()))
    acc = lax.dot_general(xc_ref[...], wc_ref[...], dn,
                          preferred_element_type=jnp.float32)
    acc = acc + lax.dot_general(xr_ref[...], wr_ref[...], dn,
                                preferred_element_type=jnp.float32)
    acc = acc + lax.dot_general(ea_ref[...], we_ref[...], dn,
                                preferred_element_type=jnp.float32)
    out_ref[...] = jnp.maximum(acc + b1_ref[...], 0.0).reshape(MLP_TILE // 8, 8, H)


def _make_edge_mlp(esize: int, ea_off: int):
    def run(xc, xr, ea, wc, wr, we, b1):
        return pl.pallas_call(
            _mlp_body,
            grid=(esize // MLP_TILE,),
            in_specs=[
                pl.BlockSpec((7, MLP_TILE), lambda i: (0, i)),
                pl.BlockSpec((7, MLP_TILE), lambda i: (0, i)),
                pl.BlockSpec((4, MLP_TILE), lambda i: (0, ea_off + i)),
                pl.BlockSpec((7, H), lambda i: (0, 0)),
                pl.BlockSpec((7, H), lambda i: (0, 0)),
                pl.BlockSpec((4, H), lambda i: (0, 0)),
                pl.BlockSpec((1, H), lambda i: (0, 0)),
            ],
            out_specs=pl.BlockSpec((MLP_TILE // 8, 8, H), lambda i: (i, 0, 0)),
            out_shape=jax.ShapeDtypeStruct((esize // 8, 8, H), jnp.float32),
        )(xc, xr, ea, wc, wr, we, b1)
    return run


_edge_mlp_a = _make_edge_mlp(E_A, 0)
_edge_mlp_b = _make_edge_mlp(E_B, E_A // MLP_TILE)


def _tc_h_body(sp_ref, deg_ref, w2_ref, b2_ref, w10_ref,
               h_ref, z_ref, dinv_ref, oacc_ref):
    sp = sp_ref[...]
    s = sp[0] + sp[1]
    deg = deg_ref[...]
    h = jnp.dot(s, w2_ref[...], preferred_element_type=jnp.float32)
    h = h + deg * b2_ref[...]
    safe = jnp.where(deg > 0, deg, 1.0)
    dinv = jnp.where(deg > 0, lax.rsqrt(safe), 0.0)
    h_ref[...] = h
    z_ref[...] = dinv * h
    dinv_ref[...] = dinv
    oacc_ref[...] = jnp.dot(h, w10_ref[...], preferred_element_type=jnp.float32)


def _tc_h(sp, deg, w2, b2, w10):
    return pl.pallas_call(
        _tc_h_body,
        grid=(N // TN,),
        in_specs=[
            pl.BlockSpec((NC, TN, H), lambda i: (0, i, 0)),
            pl.BlockSpec((TN, 1), lambda i: (i, 0)),
            pl.BlockSpec((H, H), lambda i: (0, 0)),
            pl.BlockSpec((1, H), lambda i: (0, 0)),
            pl.BlockSpec((H, H), lambda i: (0, 0)),
        ],
        out_specs=[
            pl.BlockSpec((TN, H), lambda i: (i, 0)),
            pl.BlockSpec((TN, H), lambda i: (i, 0)),
            pl.BlockSpec((TN, 1), lambda i: (i, 0)),
            pl.BlockSpec((TN, H), lambda i: (i, 0)),
        ],
        out_shape=[
            jax.ShapeDtypeStruct((N, H), jnp.float32),
            jax.ShapeDtypeStruct((N, H), jnp.float32),
            jax.ShapeDtypeStruct((N, 1), jnp.float32),
            jax.ShapeDtypeStruct((N, H), jnp.float32),
        ],
    )(sp, deg, w2, b2, w10)


def _tc_z_body(aggp_ref, dinv_ref, xk_ref, z_ref):
    aggp = aggp_ref[...]
    dinv = dinv_ref[...]
    xk = dinv * (aggp[0] + aggp[1])
    xk_ref[...] = xk
    z_ref[...] = dinv * xk


def _tc_z(aggp, dinv):
    return pl.pallas_call(
        _tc_z_body,
        grid=(N // TN,),
        in_specs=[
            pl.BlockSpec((NC, TN, H), lambda i: (0, i, 0)),
            pl.BlockSpec((TN, 1), lambda i: (i, 0)),
        ],
        out_specs=[
            pl.BlockSpec((TN, H), lambda i: (i, 0)),
            pl.BlockSpec((TN, H), lambda i: (i, 0)),
        ],
        out_shape=[
            jax.ShapeDtypeStruct((N, H), jnp.float32),
            jax.ShapeDtypeStruct((N, H), jnp.float32),
        ],
    )(aggp, dinv)


def _tc_accum_body(xk_ref, w_ref, oacc_ref, oout_ref):
    oout_ref[...] = oacc_ref[...] + jnp.dot(xk_ref[...], w_ref[...],
                                            preferred_element_type=jnp.float32)


def _tc_accum(xk, w, oacc):
    return pl.pallas_call(
        _tc_accum_body,
        grid=(N // TN,),
        in_specs=[
            pl.BlockSpec((TN, H), lambda i: (i, 0)),
            pl.BlockSpec((H, H), lambda i: (0, 0)),
            pl.BlockSpec((TN, H), lambda i: (i, 0)),
        ],
        out_specs=pl.BlockSpec((TN, H), lambda i: (i, 0)),
        out_shape=jax.ShapeDtypeStruct((N, H), jnp.float32),
    )(xk, w, oacc)


def _tc_tag_end_body(aggp_ref, dinv_ref, w_ref, oacc_ref, b_ref, wn_ref,
                     z_ref, oout_ref):
    aggp = aggp_ref[...]
    dinv = dinv_ref[...]
    xk = dinv * (aggp[0] + aggp[1])
    o = oacc_ref[...] + jnp.dot(xk, w_ref[...], preferred_element_type=jnp.float32)
    h = jnp.maximum(o + b_ref[...], 0.0)
    z_ref[...] = dinv * h
    oout_ref[...] = jnp.dot(h, wn_ref[...], preferred_element_type=jnp.float32)


def _tc_tag_end(aggp, dinv, w, oacc, b, wn):
    return pl.pallas_call(
        _tc_tag_end_body,
        grid=(N // TN,),
        in_specs=[
            pl.BlockSpec((NC, TN, H), lambda i: (0, i, 0)),
            pl.BlockSpec((TN, 1), lambda i: (i, 0)),
            pl.BlockSpec((H, H), lambda i: (0, 0)),
            pl.BlockSpec((TN, H), lambda i: (i, 0)),
            pl.BlockSpec((1, H), lambda i: (0, 0)),
            pl.BlockSpec((H, H), lambda i: (0, 0)),
        ],
        out_specs=[
            pl.BlockSpec((TN, H), lambda i: (i, 0)),
            pl.BlockSpec((TN, H), lambda i: (i, 0)),
        ],
        out_shape=[
            jax.ShapeDtypeStruct((N, H), jnp.float32),
            jax.ShapeDtypeStruct((N, H), jnp.float32),
        ],
    )(aggp, dinv, w, oacc, b, wn)


def _tc_tag_final_body(aggp_ref, dinv_ref, w_ref, oacc_ref, b_ref,
                       wo_ref, bo_ref, y_ref):
    aggp = aggp_ref[...]
    dinv = dinv_ref[...]
    xk = dinv * (aggp[0] + aggp[1])
    o = oacc_ref[...] + jnp.dot(xk, w_ref[...], preferred_element_type=jnp.float32)
    h = o + b_ref[...]
    y_ref[...] = jnp.dot(h, wo_ref[...], preferred_element_type=jnp.float32) + bo_ref[...]


def _tc_tag_final(aggp, dinv, w, oacc, b, wo, bo):
    return pl.pallas_call(
        _tc_tag_final_body,
        grid=(N // TN,),
        in_specs=[
            pl.BlockSpec((NC, TN, H), lambda i: (0, i, 0)),
            pl.BlockSpec((TN, 1), lambda i: (i, 0)),
            pl.BlockSpec((H, H), lambda i: (0, 0)),
            pl.BlockSpec((TN, H), lambda i: (i, 0)),
            pl.BlockSpec((1, H), lambda i: (0, 0)),
            pl.BlockSpec((H, 2), lambda i: (0, 0)),
            pl.BlockSpec((1, 2), lambda i: (0, 0)),
        ],
        out_specs=pl.BlockSpec((TN, 2), lambda i: (i, 0)),
        out_shape=jax.ShapeDtypeStruct((N, 2), jnp.float32),
    )(aggp, dinv, w, oacc, b, wo, bo)


def kernel(x, edge_index, edge_attr, ea_W1, ea_b1, ea_W2, ea_b2, tag_W, tag_b, out_W, out_b):
    row = edge_index[0]
    col = edge_index[1]

    npad = E_PAD - E
    pad_ids = jnp.arange(npad, dtype=jnp.int32) % NPAD
    row_p = jnp.concatenate([row, pad_ids])
    col_p = jnp.concatenate([col, N + pad_ids])
    ea_t = jnp.pad(edge_attr.T, ((0, 0), (0, npad)))

    x7 = jnp.pad(x, ((0, NPAD), (0, 0))).reshape(-1)
    zerosH = jnp.zeros((NA, H), jnp.float32)

    wc = ea_W1[0:7]
    wr = ea_W1[7:14]
    we = ea_W1[14:18]
    b1 = ea_b1.reshape(1, H)

    xcA, xrA, degpA = _sc_gather_x_a(row_p, col_p, x7)
    h1A = _edge_mlp_a(xcA, xrA, ea_t, wc, wr, we, b1).reshape(E_A, H)
    xcB, xrB, degpB = _sc_gather_x_b(row_p, col_p, x7)
    h1B = _edge_mlp_b(xcB, xrB, ea_t, wc, wr, we, b1).reshape(E_B, H)
    spA = _sc_agg_lin_a(h1A, row_p, col_p, zerosH)
    sp = _sc_agg_lin_b(h1B, row_p, col_p, spA)

    deg = (degpA + degpB).reshape(NW, NA).sum(axis=0)[:N].reshape(N, 1)

    h, z, dinv, oacc = _tc_h(sp, deg, ea_W2, ea_b2.reshape(1, H), tag_W[0, 0])

    aggp = _sc_agg_gather(z, row_p, col_p, zerosH)
    xk, z = _tc_z(aggp, dinv)
    aggp = _sc_agg_gather(z, row_p, col_p, zerosH)
    oacc = _tc_accum(xk, tag_W[0, 1], oacc)
    xk, z = _tc_z(aggp, dinv)
    aggp = _sc_agg_gather(z, row_p, col_p, zerosH)
    oacc = _tc_accum(xk, tag_W[0, 2], oacc)
    z, oacc = _tc_tag_end(aggp, dinv, tag_W[0, 3], oacc,
                          tag_b[0].reshape(1, H), tag_W[1, 0])
    aggp = _sc_agg_gather(z, row_p, col_p, zerosH)
    xk, z = _tc_z(aggp, dinv)
    aggp = _sc_agg_gather(z, row_p, col_p, zerosH)
    oacc = _tc_accum(xk, tag_W[1, 1], oacc)
    xk, z = _tc_z(aggp, dinv)
    aggp = _sc_agg_gather(z, row_p, col_p, zerosH)
    oacc = _tc_accum(xk, tag_W[1, 2], oacc)
    y = _tc_tag_final(aggp, dinv, tag_W[1, 3], oacc,
                      tag_b[1].reshape(1, H), out_W, out_b.reshape(1, 2))

    return y.reshape(1, -1)


# speedup vs baseline: 1.0358x; 1.0358x over previous
import functools

import jax
import jax.numpy as jnp
from jax import lax
from jax.experimental import pallas as pl
from jax.experimental.pallas import tpu as pltpu
from jax.experimental.pallas import tpu_sc as plsc

N = 10000
E = 640000
H = 128
NPAD = 112
NA = N + NPAD
NC = 2
NS = 16
NW = NC * NS
CW = 128
WPW = 157
W_A = 80
W_B = WPW - W_A
EPA = W_A * CW
EPB = W_B * CW
E_A = NW * EPA
E_B = NW * EPB
E_PAD = E_A + E_B
NA_PER_S = NA // NS
NBUF = 4

MLP_TILE = 4096
TN = 1000

_mesh = plsc.VectorSubcoreMesh(core_axis_name="c", subcore_axis_name="s")


def _worker_id():
    return lax.axis_index("s") * NC + lax.axis_index("c")


def _goff(wid, gw):
    return jnp.where(gw < W_A,
                     wid * EPA + gw * CW,
                     E_A + wid * EPB + (gw - W_A) * CW)


def _make_sc_gather_x(w0: int, nwin: int, esize: int, delta: int):
    epw = nwin * CW

    @functools.partial(
        pl.kernel,
        mesh=_mesh,
        out_type=[
            jax.ShapeDtypeStruct((7, esize), jnp.float32),
            jax.ShapeDtypeStruct((7, esize), jnp.float32),
            jax.ShapeDtypeStruct((NW, 1, NA), jnp.float32),
        ],
        scratch_types=[
            pltpu.VMEM((NA * 7,), jnp.float32),
            pltpu.VMEM((epw,), jnp.int32),
            pltpu.VMEM((epw,), jnp.int32),
            pltpu.VMEM((7, CW), jnp.float32),
            pltpu.VMEM((7, CW), jnp.float32),
            pltpu.VMEM((7, CW), jnp.float32),
            pltpu.VMEM((7, CW), jnp.float32),
            pltpu.VMEM((NA,), jnp.float32),
            pltpu.SemaphoreType.DMA,
            pltpu.SemaphoreType.DMA,
            pltpu.SemaphoreType.DMA,
            pltpu.SemaphoreType.DMA,
        ],
        compiler_params=pltpu.CompilerParams(needs_layout_passes=False),
    )
    def gather_x(row_hbm, col_hbm, x7_hbm,
                 xc_hbm, xr_hbm, deg_hbm,
                 x7v, cidx, ridx, cb0, cb1, rb0, rb1, dacc,
                 cs0, cs1, rs0, rs1):
        wid = _worker_id()
        base = _goff(wid, w0)
        cbs = (cb0, cb1)
        rbs = (rb0, rb1)
        css = (cs0, cs1)
        rss = (rs0, rs1)

        def zbody(i, _):
            dacc[pl.ds(i * 16, 16)] = jnp.zeros((16,), jnp.float32)
            return _

        lax.fori_loop(0, NA // 16, zbody, None)
        pltpu.sync_copy(x7_hbm, x7v)
        pltpu.sync_copy(col_hbm.at[pl.ds(base, epw)], cidx)
        pltpu.sync_copy(row_hbm.at[pl.ds(base, epw)], ridx)
        ones16 = jnp.ones((16,), jnp.float32)

        def compute(w, b):
            for j in range(CW // 16):
                c16 = cidx[pl.ds(w * CW + j * 16, 16)]
                r16 = ridx[pl.ds(w * CW + j * 16, 16)]
                plsc.addupdate_scatter(dacc, [c16], ones16)
                c7 = c16 * 7
                r7 = r16 * 7
                for f in range(7):
                    cbs[b][f, pl.ds(j * 16, 16)] = plsc.load_gather(x7v, [c7 + f])
                    rbs[b][f, pl.ds(j * 16, 16)] = plsc.load_gather(x7v, [r7 + f])

        def issue_out(w, b):
            off = base - delta + w * CW
            pltpu.async_copy(cbs[b], xc_hbm.at[:, pl.ds(off, CW)], css[b])
            pltpu.async_copy(rbs[b], xr_hbm.at[:, pl.ds(off, CW)], rss[b])

        def wait_out(w, b):
            off = base - delta + w * CW
            pltpu.make_async_copy(cbs[b], xc_hbm.at[:, pl.ds(off, CW)], css[b]).wait()
            pltpu.make_async_copy(rbs[b], xr_hbm.at[:, pl.ds(off, CW)], rss[b]).wait()

        def body(k, _):
            for b in range(2):
                w = 2 * k + b

                @pl.when(w >= 2)
                def _():
                    wait_out(w - 2, b)

                compute(w, b)
                issue_out(w, b)
            return _

        lax.fori_loop(0, nwin // 2, body, None)
        if nwin % 2:
            w = nwin - 1
            wait_out(w - 2, 0)
            compute(w, 0)
            issue_out(w, 0)
        wait_out(nwin - 2, (nwin - 2) % 2)
        wait_out(nwin - 1, (nwin - 1) % 2)
        pltpu.sync_copy(dacc, deg_hbm.at[wid, 0])

    return gather_x


_sc_gather_x_a = _make_sc_gather_x(0, W_A, E_A, 0)
_sc_gather_x_b = _make_sc_gather_x(W_A, W_B, E_B, E_A)


def _make_sc_aggregate(gather_src: bool, w0: int = 0, nwin: int = WPW,
                       src_delta: int = 0, first: bool = True):
    scratch = [pltpu.VMEM((CW,), jnp.int32)] * 7
    scratch += [pltpu.VMEM((CW, H), jnp.float32)] * 3
    scratch += [pltpu.VMEM_SHARED((NA, H), jnp.float32)]
    scratch += [pltpu.SemaphoreType.DMA] * 13

    @functools.partial(
        pl.kernel,
        mesh=_mesh,
        out_type=jax.ShapeDtypeStruct((NC, NA, H), jnp.float32),
        scratch_types=scratch,
    )
    def agg(src_hbm, row_hbm, col_hbm, init_hbm, out_hbm,
            c0, c1, c2, c3, r0, r1, r2, b0, b1, b2, acc,
            ic0, ic1, ic2, ic3, ir0, ir1, ir2, g0, g1, g2, s0, s1, s2):
        cid = lax.axis_index("c")
        sid = lax.axis_index("s")
        wid = _worker_id()
        cidx = (c0, c1, c2, c3)
        ridx = (r0, r1, r2)
        bufs = (b0, b1, b2)
        icsem = (ic0, ic1, ic2, ic3)
        irsem = (ir0, ir1, ir2)
        gsem = (g0, g1, g2)
        ssem = (s0, s1, s2)

        rows = pl.ds(sid * NA_PER_S, NA_PER_S)
        if first:
            pltpu.sync_copy(init_hbm.at[rows], acc.at[rows])
        else:
            pltpu.sync_copy(init_hbm.at[cid, rows], acc.at[rows])
        plsc.subcore_barrier()

        def off(w):
            return _goff(wid, w0 + w)

        def issue_idx(w, ic, ir):
            o = off(w)
            pltpu.async_copy(col_hbm.at[pl.ds(o, CW)], cidx[ic], icsem[ic])
            if gather_src:
                pltpu.async_copy(row_hbm.at[pl.ds(o, CW)], ridx[ir], irsem[ir])

        def wait_idx(w, ic, ir):
            o = off(w)
            pltpu.make_async_copy(col_hbm.at[pl.ds(o, CW)], cidx[ic], icsem[ic]).wait()
            if gather_src:
                pltpu.make_async_copy(row_hbm.at[pl.ds(o, CW)], ridx[ir], irsem[ir]).wait()

        def gsrc(w, ir):
            if gather_src:
                return src_hbm.at[ridx[ir]]
            return src_hbm.at[pl.ds(off(w) - src_delta, CW)]

        def issue_gather(w, ir, b):
            pltpu.async_copy(gsrc(w, ir), bufs[b], gsem[b])

        def wait_gather(w, ir, b):
            pltpu.make_async_copy(gsrc(w, ir), bufs[b], gsem[b]).wait()

        def issue_scatter(w, ic, b):
            pltpu.async_copy(bufs[b], acc.at[cidx[ic]], ssem[b], add=True)

        def wait_scatter(w, ic, b):
            pltpu.make_async_copy(bufs[b], acc.at[cidx[ic]], ssem[b]).wait()

        issue_idx(0, 0, 0)
        issue_idx(1, 1, 1)

        def slot(w, j):
            @pl.when(w >= 2)
            def _():
                wait_scatter(w - 2, (j - 2) % 4, (j - 2) % 3)

            wait_idx(w, j % 4, j % 3)
            issue_gather(w, j % 3, j % 3)

            @pl.when(w >= 1)
            def _():
                wait_gather(w - 1, (j - 1) % 3, (j - 1) % 3)
                issue_scatter(w - 1, (j - 1) % 4, (j - 1) % 3)

            @pl.when(w + 2 < nwin)
            def _():
                issue_idx(w + 2, (j + 2) % 4, (j + 2) % 3)

        def body(k, _):
            w0_ = 12 * k
            for j in range(12):
                slot(w0_ + j, j)
            return _

        nfull = nwin // 12
        lax.fori_loop(0, nfull, body, None)
        for j in range(nwin % 12):
            slot(nfull * 12 + j, j)
        w = nwin - 1
        jw = w % 12
        wait_gather(w, jw % 3, jw % 3)
        issue_scatter(w, jw % 4, jw % 3)
        wait_scatter(w - 1, (jw - 1) % 4, (jw - 1) % 3)
        wait_scatter(w, jw % 4, jw % 3)

        plsc.subcore_barrier()
        pltpu.sync_copy(acc.at[rows], out_hbm.at[cid, rows])

    return agg


_sc_agg_lin_a = _make_sc_aggregate(False, 0, W_A, 0, first=True)
_sc_agg_lin_b = _make_sc_aggregate(False, W_A, W_B, E_A, first=False)
_sc_agg_gather = _make_sc_aggregate(True)


def _mlp_body(xc_ref, xr_ref, ea_ref, wc_ref, wr_ref, we_ref, b1_ref, out_ref):
    dn = (((0,), (0,)), ((), ()))
    acc = lax.dot_general(xc_ref[...], wc_ref[...], dn,
                          preferred_element_type=jnp.float32)
    acc = acc + lax.dot_general(xr_ref[...], wr_ref[...], dn,
                                preferred_element_type=jnp.float32)
    acc = acc + lax.dot_general(ea_ref[...], we_ref[...], dn,
                                preferred_element_type=jnp.float32)
    out_ref[...] = jnp.maximum(acc + b1_ref[...], 0.0).reshape(MLP_TILE // 8, 8, H)


def _make_edge_mlp(esize: int, ea_off: int):
    def run(xc, xr, ea, wc, wr, we, b1):
        return pl.pallas_call(
            _mlp_body,
            grid=(esize // MLP_TILE,),
            in_specs=[
                pl.BlockSpec((7, MLP_TILE), lambda i: (0, i)),
                pl.BlockSpec((7, MLP_TILE), lambda i: (0, i)),
                pl.BlockSpec((4, MLP_TILE), lambda i: (0, ea_off + i)),
                pl.BlockSpec((7, H), lambda i: (0, 0)),
                pl.BlockSpec((7, H), lambda i: (0, 0)),
                pl.BlockSpec((4, H), lambda i: (0, 0)),
                pl.BlockSpec((1, H), lambda i: (0, 0)),
            ],
            out_specs=pl.BlockSpec((MLP_TILE // 8, 8, H), lambda i: (i, 0, 0)),
            out_shape=jax.ShapeDtypeStruct((esize // 8, 8, H), jnp.float32),
        )(xc, xr, ea, wc, wr, we, b1)
    return run


_edge_mlp_a = _make_edge_mlp(E_A, 0)
_edge_mlp_b = _make_edge_mlp(E_B, E_A // MLP_TILE)


def _tc_h_body(sp_ref, deg_ref, w2_ref, b2_ref, w10_ref,
               h_ref, z_ref, dinv_ref, oacc_ref):
    sp = sp_ref[...]
    s = sp[0] + sp[1]
    deg = deg_ref[...]
    h = jnp.dot(s, w2_ref[...], preferred_element_type=jnp.float32)
    h = h + deg * b2_ref[...]
    safe = jnp.where(deg > 0, deg, 1.0)
    dinv = jnp.where(deg > 0, lax.rsqrt(safe), 0.0)
    h_ref[...] = h
    z_ref[...] = dinv * h
    dinv_ref[...] = dinv
    oacc_ref[...] = jnp.dot(h, w10_ref[...], preferred_element_type=jnp.float32)


def _tc_h(sp, deg, w2, b2, w10):
    return pl.pallas_call(
        _tc_h_body,
        grid=(N // TN,),
        in_specs=[
            pl.BlockSpec((NC, TN, H), lambda i: (0, i, 0)),
            pl.BlockSpec((TN, 1), lambda i: (i, 0)),
            pl.BlockSpec((H, H), lambda i: (0, 0)),
            pl.BlockSpec((1, H), lambda i: (0, 0)),
            pl.BlockSpec((H, H), lambda i: (0, 0)),
        ],
        out_specs=[
            pl.BlockSpec((TN, H), lambda i: (i, 0)),
            pl.BlockSpec((TN, H), lambda i: (i, 0)),
            pl.BlockSpec((TN, 1), lambda i: (i, 0)),
            pl.BlockSpec((TN, H), lambda i: (i, 0)),
        ],
        out_shape=[
            jax.ShapeDtypeStruct((N, H), jnp.float32),
            jax.ShapeDtypeStruct((N, H), jnp.float32),
            jax.ShapeDtypeStruct((N, 1), jnp.float32),
            jax.ShapeDtypeStruct((N, H), jnp.float32),
        ],
    )(sp, deg, w2, b2, w10)


def _tc_z_body(aggp_ref, dinv_ref, xk_ref, z_ref):
    aggp = aggp_ref[...]
    dinv = dinv_ref[...]
    xk = dinv * (aggp[0] + aggp[1])
    xk_ref[...] = xk
    z_ref[...] = dinv * xk


def _tc_z(aggp, dinv):
    return pl.pallas_call(
        _tc_z_body,
        grid=(N // TN,),
        in_specs=[
            pl.BlockSpec((NC, TN, H), lambda i: (0, i, 0)),
            pl.BlockSpec((TN, 1), lambda i: (i, 0)),
        ],
        out_specs=[
            pl.BlockSpec((TN, H), lambda i: (i, 0)),
            pl.BlockSpec((TN, H), lambda i: (i, 0)),
        ],
        out_shape=[
            jax.ShapeDtypeStruct((N, H), jnp.float32),
            jax.ShapeDtypeStruct((N, H), jnp.float32),
        ],
    )(aggp, dinv)


def _tc_accum_body(xk_ref, w_ref, oacc_ref, oout_ref):
    oout_ref[...] = oacc_ref[...] + jnp.dot(xk_ref[...], w_ref[...],
                                            preferred_element_type=jnp.float32)


def _tc_accum(xk, w, oacc):
    return pl.pallas_call(
        _tc_accum_body,
        grid=(N // TN,),
        in_specs=[
            pl.BlockSpec((TN, H), lambda i: (i, 0)),
            pl.BlockSpec((H, H), lambda i: (0, 0)),
            pl.BlockSpec((TN, H), lambda i: (i, 0)),
        ],
        out_specs=pl.BlockSpec((TN, H), lambda i: (i, 0)),
        out_shape=jax.ShapeDtypeStruct((N, H), jnp.float32),
    )(xk, w, oacc)


def _tc_tag_end_body(aggp_ref, dinv_ref, w_ref, oacc_ref, b_ref, wn_ref,
                     z_ref, oout_ref):
    aggp = aggp_ref[...]
    dinv = dinv_ref[...]
    xk = dinv * (aggp[0] + aggp[1])
    o = oacc_ref[...] + jnp.dot(xk, w_ref[...], preferred_element_type=jnp.float32)
    h = jnp.maximum(o + b_ref[...], 0.0)
    z_ref[...] = dinv * h
    oout_ref[...] = jnp.dot(h, wn_ref[...], preferred_element_type=jnp.float32)


def _tc_tag_end(aggp, dinv, w, oacc, b, wn):
    return pl.pallas_call(
        _tc_tag_end_body,
        grid=(N // TN,),
        in_specs=[
            pl.BlockSpec((NC, TN, H), lambda i: (0, i, 0)),
            pl.BlockSpec((TN, 1), lambda i: (i, 0)),
            pl.BlockSpec((H, H), lambda i: (0, 0)),
            pl.BlockSpec((TN, H), lambda i: (i, 0)),
            pl.BlockSpec((1, H), lambda i: (0, 0)),
            pl.BlockSpec((H, H), lambda i: (0, 0)),
        ],
        out_specs=[
            pl.BlockSpec((TN, H), lambda i: (i, 0)),
            pl.BlockSpec((TN, H), lambda i: (i, 0)),
        ],
        out_shape=[
            jax.ShapeDtypeStruct((N, H), jnp.float32),
            jax.ShapeDtypeStruct((N, H), jnp.float32),
        ],
    )(aggp, dinv, w, oacc, b, wn)


def _tc_tag_final_body(aggp_ref, dinv_ref, w_ref, oacc_ref, b_ref,
                       wo_ref, bo_ref, y_ref):
    aggp = aggp_ref[...]
    dinv = dinv_ref[...]
    xk = dinv * (aggp[0] + aggp[1])
    o = oacc_ref[...] + jnp.dot(xk, w_ref[...], preferred_element_type=jnp.float32)
    h = o + b_ref[...]
    y_ref[...] = jnp.dot(h, wo_ref[...], preferred_element_type=jnp.float32) + bo_ref[...]


def _tc_tag_final(aggp, dinv, w, oacc, b, wo, bo):
    return pl.pallas_call(
        _tc_tag_final_body,
        grid=(N // TN,),
        in_specs=[
            pl.BlockSpec((NC, TN, H), lambda i: (0, i, 0)),
            pl.BlockSpec((TN, 1), lambda i: (i, 0)),
            pl.BlockSpec((H, H), lambda i: (0, 0)),
            pl.BlockSpec((TN, H), lambda i: (i, 0)),
            pl.BlockSpec((1, H), lambda i: (0, 0)),
            pl.BlockSpec((H, 2), lambda i: (0, 0)),
            pl.BlockSpec((1, 2), lambda i: (0, 0)),
        ],
        out_specs=pl.BlockSpec((TN, 2), lambda i: (i, 0)),
        out_shape=jax.ShapeDtypeStruct((N, 2), jnp.float32),
    )(aggp, dinv, w, oacc, b, wo, bo)


def kernel(x, edge_index, edge_attr, ea_W1, ea_b1, ea_W2, ea_b2, tag_W, tag_b, out_W, out_b):
    row = edge_index[0]
    col = edge_index[1]

    npad = E_PAD - E
    pad_ids = jnp.arange(npad, dtype=jnp.int32) % NPAD
    row_p = jnp.concatenate([row, pad_ids])
    col_p = jnp.concatenate([col, N + pad_ids])
    ea_t = jnp.pad(edge_attr.T, ((0, 0), (0, npad)))

    x7 = jnp.pad(x, ((0, NPAD), (0, 0))).reshape(-1)
    zerosH = jnp.zeros((NA, H), jnp.float32)

    wc = ea_W1[0:7]
    wr = ea_W1[7:14]
    we = ea_W1[14:18]
    b1 = ea_b1.reshape(1, H)

    xcA, xrA, degpA = _sc_gather_x_a(row_p, col_p, x7)
    h1A = _edge_mlp_a(xcA, xrA, ea_t, wc, wr, we, b1).reshape(E_A, H)
    xcB, xrB, degpB = _sc_gather_x_b(row_p, col_p, x7)
    h1B = _edge_mlp_b(xcB, xrB, ea_t, wc, wr, we, b1).reshape(E_B, H)
    spA = _sc_agg_lin_a(h1A, row_p, col_p, zerosH)
    sp = _sc_agg_lin_b(h1B, row_p, col_p, spA)

    deg = (degpA + degpB).reshape(NW, NA).sum(axis=0)[:N].reshape(N, 1)

    h, z, dinv, oacc = _tc_h(sp, deg, ea_W2, ea_b2.reshape(1, H), tag_W[0, 0])

    aggp = _sc_agg_gather(z, row_p, col_p, zerosH)
    xk, z = _tc_z(aggp, dinv)
    aggp = _sc_agg_gather(z, row_p, col_p, zerosH)
    oacc = _tc_accum(xk, tag_W[0, 1], oacc)
    xk, z = _tc_z(aggp, dinv)
    aggp = _sc_agg_gather(z, row_p, col_p, zerosH)
    oacc = _tc_accum(xk, tag_W[0, 2], oacc)
    z, oacc = _tc_tag_end(aggp, dinv, tag_W[0, 3], oacc,
                          tag_b[0].reshape(1, H), tag_W[1, 0])
    aggp = _sc_agg_gather(z, row_p, col_p, zerosH)
    xk, z = _tc_z(aggp, dinv)
    aggp = _sc_agg_gather(z, row_p, col_p, zerosH)
    oacc = _tc_accum(xk, tag_W[1, 1], oacc)
    xk, z = _tc_z(aggp, dinv)
    aggp = _sc_agg_gather(z, row_p, col_p, zerosH)
    oacc = _tc_accum(xk, tag_W[1, 2], oacc)
    y = _tc_tag_final(aggp, dinv, tag_W[1, 3], oacc,
                      tag_b[1].reshape(1, H), out_W, out_b.reshape(1, 2))

    return y.reshape(1, -1)


# speedup vs baseline: 1.0428x; 1.0068x over previous
import functools

import jax
import jax.numpy as jnp
from jax import lax
from jax.experimental import pallas as pl
from jax.experimental.pallas import tpu as pltpu
from jax.experimental.pallas import tpu_sc as plsc

N = 10000
E = 640000
H = 128
NPAD = 112
NA = N + NPAD
NC = 2
NS = 16
NW = NC * NS
CW = 128
WPW = 157
W_A = 80
W_B = WPW - W_A
EPA = W_A * CW
EPB = W_B * CW
E_A = NW * EPA
E_B = NW * EPB
E_PAD = E_A + E_B
NA_PER_S = NA // NS
NBUF = 4

MLP_TILE = 4096
TN = 2000

_mesh = plsc.VectorSubcoreMesh(core_axis_name="c", subcore_axis_name="s")


def _worker_id():
    return lax.axis_index("s") * NC + lax.axis_index("c")


def _goff(wid, gw):
    return jnp.where(gw < W_A,
                     wid * EPA + gw * CW,
                     E_A + wid * EPB + (gw - W_A) * CW)


def _make_sc_gather_x(w0: int, nwin: int, esize: int, delta: int):
    epw = nwin * CW

    @functools.partial(
        pl.kernel,
        mesh=_mesh,
        out_type=[
            jax.ShapeDtypeStruct((7, esize), jnp.float32),
            jax.ShapeDtypeStruct((7, esize), jnp.float32),
            jax.ShapeDtypeStruct((NW, 1, NA), jnp.float32),
        ],
        scratch_types=[
            pltpu.VMEM((NA * 7,), jnp.float32),
            pltpu.VMEM((epw,), jnp.int32),
            pltpu.VMEM((epw,), jnp.int32),
            pltpu.VMEM((7, CW), jnp.float32),
            pltpu.VMEM((7, CW), jnp.float32),
            pltpu.VMEM((7, CW), jnp.float32),
            pltpu.VMEM((7, CW), jnp.float32),
            pltpu.VMEM((NA,), jnp.float32),
            pltpu.SemaphoreType.DMA,
            pltpu.SemaphoreType.DMA,
            pltpu.SemaphoreType.DMA,
            pltpu.SemaphoreType.DMA,
        ],
        compiler_params=pltpu.CompilerParams(needs_layout_passes=False),
    )
    def gather_x(row_hbm, col_hbm, x7_hbm,
                 xc_hbm, xr_hbm, deg_hbm,
                 x7v, cidx, ridx, cb0, cb1, rb0, rb1, dacc,
                 cs0, cs1, rs0, rs1):
        wid = _worker_id()
        base = _goff(wid, w0)
        cbs = (cb0, cb1)
        rbs = (rb0, rb1)
        css = (cs0, cs1)
        rss = (rs0, rs1)

        def zbody(i, _):
            dacc[pl.ds(i * 16, 16)] = jnp.zeros((16,), jnp.float32)
            return _

        lax.fori_loop(0, NA // 16, zbody, None)
        pltpu.sync_copy(x7_hbm, x7v)
        pltpu.sync_copy(col_hbm.at[pl.ds(base, epw)], cidx)
        pltpu.sync_copy(row_hbm.at[pl.ds(base, epw)], ridx)
        ones16 = jnp.ones((16,), jnp.float32)

        def compute(w, b):
            for j in range(CW // 16):
                c16 = cidx[pl.ds(w * CW + j * 16, 16)]
                r16 = ridx[pl.ds(w * CW + j * 16, 16)]
                plsc.addupdate_scatter(dacc, [c16], ones16)
                c7 = c16 * 7
                r7 = r16 * 7
                for f in range(7):
                    cbs[b][f, pl.ds(j * 16, 16)] = plsc.load_gather(x7v, [c7 + f])
                    rbs[b][f, pl.ds(j * 16, 16)] = plsc.load_gather(x7v, [r7 + f])

        def issue_out(w, b):
            off = base - delta + w * CW
            pltpu.async_copy(cbs[b], xc_hbm.at[:, pl.ds(off, CW)], css[b])
            pltpu.async_copy(rbs[b], xr_hbm.at[:, pl.ds(off, CW)], rss[b])

        def wait_out(w, b):
            off = base - delta + w * CW
            pltpu.make_async_copy(cbs[b], xc_hbm.at[:, pl.ds(off, CW)], css[b]).wait()
            pltpu.make_async_copy(rbs[b], xr_hbm.at[:, pl.ds(off, CW)], rss[b]).wait()

        def body(k, _):
            for b in range(2):
                w = 2 * k + b

                @pl.when(w >= 2)
                def _():
                    wait_out(w - 2, b)

                compute(w, b)
                issue_out(w, b)
            return _

        lax.fori_loop(0, nwin // 2, body, None)
        if nwin % 2:
            w = nwin - 1
            wait_out(w - 2, 0)
            compute(w, 0)
            issue_out(w, 0)
        wait_out(nwin - 2, (nwin - 2) % 2)
        wait_out(nwin - 1, (nwin - 1) % 2)
        pltpu.sync_copy(dacc, deg_hbm.at[wid, 0])

    return gather_x


_sc_gather_x_a = _make_sc_gather_x(0, W_A, E_A, 0)
_sc_gather_x_b = _make_sc_gather_x(W_A, W_B, E_B, E_A)


def _make_sc_aggregate(gather_src: bool, w0: int = 0, nwin: int = WPW,
                       src_delta: int = 0, first: bool = True):
    scratch = [pltpu.VMEM((CW,), jnp.int32)] * 7
    scratch += [pltpu.VMEM((CW, H), jnp.float32)] * 3
    scratch += [pltpu.VMEM_SHARED((NA, H), jnp.float32)]
    scratch += [pltpu.SemaphoreType.DMA] * 13

    @functools.partial(
        pl.kernel,
        mesh=_mesh,
        out_type=jax.ShapeDtypeStruct((NC, NA, H), jnp.float32),
        scratch_types=scratch,
    )
    def agg(src_hbm, row_hbm, col_hbm, init_hbm, out_hbm,
            c0, c1, c2, c3, r0, r1, r2, b0, b1, b2, acc,
            ic0, ic1, ic2, ic3, ir0, ir1, ir2, g0, g1, g2, s0, s1, s2):
        cid = lax.axis_index("c")
        sid = lax.axis_index("s")
        wid = _worker_id()
        cidx = (c0, c1, c2, c3)
        ridx = (r0, r1, r2)
        bufs = (b0, b1, b2)
        icsem = (ic0, ic1, ic2, ic3)
        irsem = (ir0, ir1, ir2)
        gsem = (g0, g1, g2)
        ssem = (s0, s1, s2)

        rows = pl.ds(sid * NA_PER_S, NA_PER_S)
        if first:
            pltpu.sync_copy(init_hbm.at[rows], acc.at[rows])
        else:
            pltpu.sync_copy(init_hbm.at[cid, rows], acc.at[rows])
        plsc.subcore_barrier()

        def off(w):
            return _goff(wid, w0 + w)

        def issue_idx(w, ic, ir):
            o = off(w)
            pltpu.async_copy(col_hbm.at[pl.ds(o, CW)], cidx[ic], icsem[ic])
            if gather_src:
                pltpu.async_copy(row_hbm.at[pl.ds(o, CW)], ridx[ir], irsem[ir])

        def wait_idx(w, ic, ir):
            o = off(w)
            pltpu.make_async_copy(col_hbm.at[pl.ds(o, CW)], cidx[ic], icsem[ic]).wait()
            if gather_src:
                pltpu.make_async_copy(row_hbm.at[pl.ds(o, CW)], ridx[ir], irsem[ir]).wait()

        def gsrc(w, ir):
            if gather_src:
                return src_hbm.at[ridx[ir]]
            return src_hbm.at[pl.ds(off(w) - src_delta, CW)]

        def issue_gather(w, ir, b):
            pltpu.async_copy(gsrc(w, ir), bufs[b], gsem[b])

        def wait_gather(w, ir, b):
            pltpu.make_async_copy(gsrc(w, ir), bufs[b], gsem[b]).wait()

        def issue_scatter(w, ic, b):
            pltpu.async_copy(bufs[b], acc.at[cidx[ic]], ssem[b], add=True)

        def wait_scatter(w, ic, b):
            pltpu.make_async_copy(bufs[b], acc.at[cidx[ic]], ssem[b]).wait()

        issue_idx(0, 0, 0)
        issue_idx(1, 1, 1)

        def slot(w, j):
            @pl.when(w >= 2)
            def _():
                wait_scatter(w - 2, (j - 2) % 4, (j - 2) % 3)

            wait_idx(w, j % 4, j % 3)
            issue_gather(w, j % 3, j % 3)

            @pl.when(w >= 1)
            def _():
                wait_gather(w - 1, (j - 1) % 3, (j - 1) % 3)
                issue_scatter(w - 1, (j - 1) % 4, (j - 1) % 3)

            @pl.when(w + 2 < nwin)
            def _():
                issue_idx(w + 2, (j + 2) % 4, (j + 2) % 3)

        def body(k, _):
            w0_ = 12 * k
            for j in range(12):
                slot(w0_ + j, j)
            return _

        nfull = nwin // 12
        lax.fori_loop(0, nfull, body, None)
        for j in range(nwin % 12):
            slot(nfull * 12 + j, j)
        w = nwin - 1
        jw = w % 12
        wait_gather(w, jw % 3, jw % 3)
        issue_scatter(w, jw % 4, jw % 3)
        wait_scatter(w - 1, (jw - 1) % 4, (jw - 1) % 3)
        wait_scatter(w, jw % 4, jw % 3)

        plsc.subcore_barrier()
        pltpu.sync_copy(acc.at[rows], out_hbm.at[cid, rows])

    return agg


_sc_agg_lin_a = _make_sc_aggregate(False, 0, W_A, 0, first=True)
_sc_agg_lin_b = _make_sc_aggregate(False, W_A, W_B, E_A, first=False)
_sc_agg_gather = _make_sc_aggregate(True)


def _mlp_body(xc_ref, xr_ref, ea_ref, wc_ref, wr_ref, we_ref, b1_ref, out_ref):
    dn = (((0,), (0,)), ((), ()))
    acc = lax.dot_general(xc_ref[...], wc_ref[...], dn,
                          preferred_element_type=jnp.float32)
    acc = acc + lax.dot_general(xr_ref[...], wr_ref[...], dn,
                                preferred_element_type=jnp.float32)
    acc = acc + lax.dot_general(ea_ref[...], we_ref[...], dn,
                                preferred_element_type=jnp.float32)
    out_ref[...] = jnp.maximum(acc + b1_ref[...], 0.0).reshape(MLP_TILE // 8, 8, H)


def _make_edge_mlp(esize: int, ea_off: int):
    def run(xc, xr, ea, wc, wr, we, b1):
        return pl.pallas_call(
            _mlp_body,
            grid=(esize // MLP_TILE,),
            in_specs=[
                pl.BlockSpec((7, MLP_TILE), lambda i: (0, i)),
                pl.BlockSpec((7, MLP_TILE), lambda i: (0, i)),
                pl.BlockSpec((4, MLP_TILE), lambda i: (0, ea_off + i)),
                pl.BlockSpec((7, H), lambda i: (0, 0)),
                pl.BlockSpec((7, H), lambda i: (0, 0)),
                pl.BlockSpec((4, H), lambda i: (0, 0)),
                pl.BlockSpec((1, H), lambda i: (0, 0)),
            ],
            out_specs=pl.BlockSpec((MLP_TILE // 8, 8, H), lambda i: (i, 0, 0)),
            out_shape=jax.ShapeDtypeStruct((esize // 8, 8, H), jnp.float32),
        )(xc, xr, ea, wc, wr, we, b1)
    return run


_edge_mlp_a = _make_edge_mlp(E_A, 0)
_edge_mlp_b = _make_edge_mlp(E_B, E_A // MLP_TILE)


def _tc_h_body(sp_ref, deg_ref, w2_ref, b2_ref, w10_ref,
               h_ref, z_ref, dinv_ref, oacc_ref):
    sp = sp_ref[...]
    s = sp[0] + sp[1]
    deg = deg_ref[...]
    h = jnp.dot(s, w2_ref[...], preferred_element_type=jnp.float32)
    h = h + deg * b2_ref[...]
    safe = jnp.where(deg > 0, deg, 1.0)
    dinv = jnp.where(deg > 0, lax.rsqrt(safe), 0.0)
    h_ref[...] = h
    z_ref[...] = dinv * h
    dinv_ref[...] = dinv
    oacc_ref[...] = jnp.dot(h, w10_ref[...], preferred_element_type=jnp.float32)


def _tc_h(sp, deg, w2, b2, w10):
    return pl.pallas_call(
        _tc_h_body,
        grid=(N // TN,),
        in_specs=[
            pl.BlockSpec((NC, TN, H), lambda i: (0, i, 0)),
            pl.BlockSpec((TN, 1), lambda i: (i, 0)),
            pl.BlockSpec((H, H), lambda i: (0, 0)),
            pl.BlockSpec((1, H), lambda i: (0, 0)),
            pl.BlockSpec((H, H), lambda i: (0, 0)),
        ],
        out_specs=[
            pl.BlockSpec((TN, H), lambda i: (i, 0)),
            pl.BlockSpec((TN, H), lambda i: (i, 0)),
            pl.BlockSpec((TN, 1), lambda i: (i, 0)),
            pl.BlockSpec((TN, H), lambda i: (i, 0)),
        ],
        out_shape=[
            jax.ShapeDtypeStruct((N, H), jnp.float32),
            jax.ShapeDtypeStruct((N, H), jnp.float32),
            jax.ShapeDtypeStruct((N, 1), jnp.float32),
            jax.ShapeDtypeStruct((N, H), jnp.float32),
        ],
    )(sp, deg, w2, b2, w10)


def _tc_z_body(aggp_ref, dinv_ref, xk_ref, z_ref):
    aggp = aggp_ref[...]
    dinv = dinv_ref[...]
    xk = dinv * (aggp[0] + aggp[1])
    xk_ref[...] = xk
    z_ref[...] = dinv * xk


def _tc_z(aggp, dinv):
    return pl.pallas_call(
        _tc_z_body,
        grid=(N // TN,),
        in_specs=[
            pl.BlockSpec((NC, TN, H), lambda i: (0, i, 0)),
            pl.BlockSpec((TN, 1), lambda i: (i, 0)),
        ],
        out_specs=[
            pl.BlockSpec((TN, H), lambda i: (i, 0)),
            pl.BlockSpec((TN, H), lambda i: (i, 0)),
        ],
        out_shape=[
            jax.ShapeDtypeStruct((N, H), jnp.float32),
            jax.ShapeDtypeStruct((N, H), jnp.float32),
        ],
    )(aggp, dinv)


def _tc_accum_body(xk_ref, w_ref, oacc_ref, oout_ref):
    oout_ref[...] = oacc_ref[...] + jnp.dot(xk_ref[...], w_ref[...],
                                            preferred_element_type=jnp.float32)


def _tc_accum(xk, w, oacc):
    return pl.pallas_call(
        _tc_accum_body,
        grid=(N // TN,),
        in_specs=[
            pl.BlockSpec((TN, H), lambda i: (i, 0)),
            pl.BlockSpec((H, H), lambda i: (0, 0)),
            pl.BlockSpec((TN, H), lambda i: (i, 0)),
        ],
        out_specs=pl.BlockSpec((TN, H), lambda i: (i, 0)),
        out_shape=jax.ShapeDtypeStruct((N, H), jnp.float32),
    )(xk, w, oacc)


def _tc_tag_end_body(aggp_ref, dinv_ref, w_ref, oacc_ref, b_ref, wn_ref,
                     z_ref, oout_ref):
    aggp = aggp_ref[...]
    dinv = dinv_ref[...]
    xk = dinv * (aggp[0] + aggp[1])
    o = oacc_ref[...] + jnp.dot(xk, w_ref[...], preferred_element_type=jnp.float32)
    h = jnp.maximum(o + b_ref[...], 0.0)
    z_ref[...] = dinv * h
    oout_ref[...] = jnp.dot(h, wn_ref[...], preferred_element_type=jnp.float32)


def _tc_tag_end(aggp, dinv, w, oacc, b, wn):
    return pl.pallas_call(
        _tc_tag_end_body,
        grid=(N // TN,),
        in_specs=[
            pl.BlockSpec((NC, TN, H), lambda i: (0, i, 0)),
            pl.BlockSpec((TN, 1), lambda i: (i, 0)),
            pl.BlockSpec((H, H), lambda i: (0, 0)),
            pl.BlockSpec((TN, H), lambda i: (i, 0)),
            pl.BlockSpec((1, H), lambda i: (0, 0)),
            pl.BlockSpec((H, H), lambda i: (0, 0)),
        ],
        out_specs=[
            pl.BlockSpec((TN, H), lambda i: (i, 0)),
            pl.BlockSpec((TN, H), lambda i: (i, 0)),
        ],
        out_shape=[
            jax.ShapeDtypeStruct((N, H), jnp.float32),
            jax.ShapeDtypeStruct((N, H), jnp.float32),
        ],
    )(aggp, dinv, w, oacc, b, wn)


def _tc_tag_final_body(aggp_ref, dinv_ref, w_ref, oacc_ref, b_ref,
                       wo_ref, bo_ref, y_ref):
    aggp = aggp_ref[...]
    dinv = dinv_ref[...]
    xk = dinv * (aggp[0] + aggp[1])
    o = oacc_ref[...] + jnp.dot(xk, w_ref[...], preferred_element_type=jnp.float32)
    h = o + b_ref[...]
    y_ref[...] = jnp.dot(h, wo_ref[...], preferred_element_type=jnp.float32) + bo_ref[...]


def _tc_tag_final(aggp, dinv, w, oacc, b, wo, bo):
    return pl.pallas_call(
        _tc_tag_final_body,
        grid=(N // TN,),
        in_specs=[
            pl.BlockSpec((NC, TN, H), lambda i: (0, i, 0)),
            pl.BlockSpec((TN, 1), lambda i: (i, 0)),
            pl.BlockSpec((H, H), lambda i: (0, 0)),
            pl.BlockSpec((TN, H), lambda i: (i, 0)),
            pl.BlockSpec((1, H), lambda i: (0, 0)),
            pl.BlockSpec((H, 2), lambda i: (0, 0)),
            pl.BlockSpec((1, 2), lambda i: (0, 0)),
        ],
        out_specs=pl.BlockSpec((TN, 2), lambda i: (i, 0)),
        out_shape=jax.ShapeDtypeStruct((N, 2), jnp.float32),
    )(aggp, dinv, w, oacc, b, wo, bo)


def kernel(x, edge_index, edge_attr, ea_W1, ea_b1, ea_W2, ea_b2, tag_W, tag_b, out_W, out_b):
    row = edge_index[0]
    col = edge_index[1]

    npad = E_PAD - E
    pad_ids = jnp.arange(npad, dtype=jnp.int32) % NPAD
    row_p = jnp.concatenate([row, pad_ids])
    col_p = jnp.concatenate([col, N + pad_ids])
    ea_t = jnp.pad(edge_attr.T, ((0, 0), (0, npad)))

    x7 = jnp.pad(x, ((0, NPAD), (0, 0))).reshape(-1)
    zerosH = jnp.zeros((NA, H), jnp.float32)

    wc = ea_W1[0:7]
    wr = ea_W1[7:14]
    we = ea_W1[14:18]
    b1 = ea_b1.reshape(1, H)

    xcA, xrA, degpA = _sc_gather_x_a(row_p, col_p, x7)
    h1A = _edge_mlp_a(xcA, xrA, ea_t, wc, wr, we, b1).reshape(E_A, H)
    xcB, xrB, degpB = _sc_gather_x_b(row_p, col_p, x7)
    h1B = _edge_mlp_b(xcB, xrB, ea_t, wc, wr, we, b1).reshape(E_B, H)
    spA = _sc_agg_lin_a(h1A, row_p, col_p, zerosH)
    sp = _sc_agg_lin_b(h1B, row_p, col_p, spA)

    deg = (degpA + degpB).reshape(NW, NA).sum(axis=0)[:N].reshape(N, 1)

    h, z, dinv, oacc = _tc_h(sp, deg, ea_W2, ea_b2.reshape(1, H), tag_W[0, 0])

    aggp = _sc_agg_gather(z, row_p, col_p, zerosH)
    xk, z = _tc_z(aggp, dinv)
    aggp = _sc_agg_gather(z, row_p, col_p, zerosH)
    oacc = _tc_accum(xk, tag_W[0, 1], oacc)
    xk, z = _tc_z(aggp, dinv)
    aggp = _sc_agg_gather(z, row_p, col_p, zerosH)
    oacc = _tc_accum(xk, tag_W[0, 2], oacc)
    z, oacc = _tc_tag_end(aggp, dinv, tag_W[0, 3], oacc,
                          tag_b[0].reshape(1, H), tag_W[1, 0])
    aggp = _sc_agg_gather(z, row_p, col_p, zerosH)
    xk, z = _tc_z(aggp, dinv)
    aggp = _sc_agg_gather(z, row_p, col_p, zerosH)
    oacc = _tc_accum(xk, tag_W[1, 1], oacc)
    xk, z = _tc_z(aggp, dinv)
    aggp = _sc_agg_gather(z, row_p, col_p, zerosH)
    oacc = _tc_accum(xk, tag_W[1, 2], oacc)
    y = _tc_tag_final(aggp, dinv, tag_W[1, 3], oacc,
                      tag_b[1].reshape(1, H), out_W, out_b.reshape(1, 2))

    return y.reshape(1, -1)
